# baseline (speedup 1.0000x reference)
import functools

import jax
import jax.numpy as jnp
from jax import lax
from jax.experimental import pallas as pl
from jax.experimental.pallas import tpu as pltpu
from jax.experimental.pallas import tpu_sc as plsc

D_HALF = 64
D_MODEL = 128
N_SPLIT = 4


@functools.lru_cache(maxsize=None)
def _make_sc_gather(n_rows: int, chunk: int):
    info = plsc.get_sparse_core_info()
    nc, ns = info.num_cores, info.num_subcores
    nw = nc * ns
    n_per = n_rows // nw
    n_chunks = n_per // chunk
    assert n_per % chunk == 0 and n_rows % nw == 0 and chunk % 8 == 0

    mesh = plsc.VectorSubcoreMesh(core_axis_name="c", subcore_axis_name="s")

    @functools.partial(
        pl.kernel,
        mesh=mesh,
        compiler_params=pltpu.CompilerParams(use_tc_tiling_on_sc=False),
        out_type=jax.ShapeDtypeStruct((n_rows, D_MODEL), jnp.float32),
        scratch_types=[
            pltpu.VMEM((chunk,), jnp.int32),
            pltpu.VMEM((chunk, D_HALF), jnp.float32),
            pltpu.SemaphoreType.DMA,
        ],
    )
    def gather_k(ids_hbm, table_hbm, out_hbm, idx_v, rows_v, sem):
        wid = lax.axis_index("s") * nc + lax.axis_index("c")
        base = wid * n_per

        def body(i, carry):
            off = base + i * chunk
            pltpu.sync_copy(ids_hbm.at[pl.ds(off, chunk)], idx_v)
            pltpu.async_copy(table_hbm.at[idx_v], rows_v, sem).wait()
            pltpu.sync_copy(rows_v,
                            out_hbm.at[pl.ds(off, chunk), pl.ds(0, D_HALF)])
            return carry

        lax.fori_loop(0, n_chunks, body, 0)

    return gather_k


def _tc_body(l, id_ref, st_ref, wst_ref, bst_ref, wc_ref, bc_ref, *rest):
    o_ref = rest[-1]
    bb = st_ref.shape[0]
    stats = st_ref[...].reshape(bb * l, 10)
    pre = jnp.dot(stats, wst_ref[...], preferred_element_type=jnp.float32)
    pre = pre + bst_ref[...]
    stat_emb = 0.5 * pre * (1.0 + lax.erf(pre * 0.7071067811865476))
    id_emb = id_ref[...][:, :D_HALF]
    acc = jnp.dot(id_emb, wc_ref[:D_HALF, :],
                  preferred_element_type=jnp.float32)
    acc = acc + jnp.dot(stat_emb, wc_ref[D_HALF:, :],
                        preferred_element_type=jnp.float32)
    o_ref[...] = (acc + bc_ref[...]).reshape(bb, l, D_MODEL)


@functools.lru_cache(maxsize=None)
def _make_tc_dense(b: int, l: int, bb: int, q: int, nq: int, alias: bool):
    bq = b // nq
    assert b % nq == 0 and bq % bb == 0
    grid = (bq // bb,)
    blk = bb * l
    base = q * (bq // bb)
    in_specs = [
        pl.BlockSpec((blk, D_MODEL), lambda i: (i, 0)),
        pl.BlockSpec((bb, l, 10), lambda i: (base + i, 0, 0)),
        pl.BlockSpec((10, D_HALF), lambda i: (0, 0)),
        pl.BlockSpec((1, D_HALF), lambda i: (0, 0)),
        pl.BlockSpec((D_MODEL, D_MODEL), lambda i: (0, 0)),
        pl.BlockSpec((1, D_MODEL), lambda i: (0, 0)),
    ]
    if alias:
        in_specs.append(pl.BlockSpec(memory_space=pl.ANY))
    return pl.pallas_call(
        functools.partial(_tc_body, l),
        grid=grid,
        in_specs=in_specs,
        out_specs=pl.BlockSpec((bb, l, D_MODEL), lambda i: (base + i, 0, 0)),
        out_shape=jax.ShapeDtypeStruct((b, l, D_MODEL), jnp.float32),
        input_output_aliases={6: 0} if alias else {},
    )


def kernel(card_ids, card_stats, emb_table, W_stat, b_stat, W_comb, b_comb):
    b, l = card_ids.shape
    n_rows = b * l
    nq = N_SPLIT
    rows_q = n_rows // nq

    flat_ids = card_ids.reshape(n_rows).astype(jnp.int32) * 2
    table2 = jnp.pad(emb_table, ((0, 0), (0, D_MODEL - D_HALF)))
    table2 = table2.reshape(2 * table2.shape[0], D_HALF)

    gather = _make_sc_gather(rows_q, 800)
    id_emb_q = [
        gather(lax.slice(flat_ids, (q * rows_q,), ((q + 1) * rows_q,)), table2)
        for q in range(nq)
    ]

    bst = b_stat.reshape(1, D_HALF)
    bc = b_comb.reshape(1, D_MODEL)
    out = _make_tc_dense(b, l, 64, 0, nq, False)(
        id_emb_q[0], card_stats, W_stat, bst, W_comb, bc)
    for q in range(1, nq):
        out = _make_tc_dense(b, l, 64, q, nq, True)(
            id_emb_q[q], card_stats, W_stat, bst, W_comb, bc, out)
    return out


# speedup vs baseline: 1.6578x; 1.6578x over previous
import functools

import jax
import jax.numpy as jnp
from jax import lax
from jax.experimental import pallas as pl
from jax.experimental.pallas import tpu as pltpu
from jax.experimental.pallas import tpu_sc as plsc

D_HALF = 64
D_MODEL = 128
N_SPLIT = 5


@functools.lru_cache(maxsize=None)
def _make_sc_gather(n_rows: int, chunk: int):
    info = plsc.get_sparse_core_info()
    nc, ns = info.num_cores, info.num_subcores
    nw = nc * ns
    n_per = n_rows // nw
    n_chunks = n_per // chunk
    assert n_per % chunk == 0 and n_rows % nw == 0 and chunk % 8 == 0

    mesh = plsc.VectorSubcoreMesh(core_axis_name="c", subcore_axis_name="s")

    @functools.partial(
        pl.kernel,
        mesh=mesh,
        compiler_params=pltpu.CompilerParams(use_tc_tiling_on_sc=False),
        out_type=jax.ShapeDtypeStruct((n_rows, D_MODEL), jnp.float32),
        scratch_types=[
            pltpu.VMEM((chunk,), jnp.int32),
            pltpu.VMEM((chunk, D_HALF), jnp.float32),
            pltpu.SemaphoreType.DMA,
        ],
    )
    def gather_k(ids_hbm, table_hbm, out_hbm, idx_v, rows_v, sem):
        wid = lax.axis_index("s") * nc + lax.axis_index("c")
        base = wid * n_per

        def body(i, carry):
            off = base + i * chunk
            pltpu.sync_copy(ids_hbm.at[pl.ds(off, chunk)], idx_v)
            pltpu.async_copy(table_hbm.at[idx_v], rows_v, sem).wait()
            pltpu.sync_copy(rows_v,
                            out_hbm.at[pl.ds(off, chunk), pl.ds(0, D_HALF)])
            return carry

        lax.fori_loop(0, n_chunks, body, 0)

    return gather_k


def _tc_body(id_ref, st_ref, wst_ref, bst_ref, wc_ref, bc_ref, *rest):
    o_ref = rest[-1]
    n = id_ref.shape[0]
    st = st_ref[...].reshape(10, n)
    pre = lax.dot_general(st, wst_ref[...], (((0,), (0,)), ((), ())),
                          preferred_element_type=jnp.float32)
    pre = pre + bst_ref[...]
    stat_emb = 0.5 * pre * (1.0 + lax.erf(pre * 0.7071067811865476))
    id_emb = id_ref[...][:, :D_HALF]
    acc = jnp.dot(id_emb, wc_ref[:D_HALF, :],
                  preferred_element_type=jnp.float32)
    acc = acc + jnp.dot(stat_emb, wc_ref[D_HALF:, :],
                        preferred_element_type=jnp.float32)
    o_ref[...] = (acc + bc_ref[...]).reshape(1, n, D_MODEL)


@functools.lru_cache(maxsize=None)
def _make_tc_dense(b: int, l: int, s: int, ns: int, alias: bool):
    lq = l // ns
    assert l % ns == 0
    grid = (lq,)
    base = s * lq
    in_specs = [
        pl.BlockSpec((b, D_MODEL), lambda j: (base + j, 0)),
        pl.BlockSpec((1, 10, b), lambda j: (base + j, 0, 0)),
        pl.BlockSpec((10, D_HALF), lambda j: (0, 0)),
        pl.BlockSpec((1, D_HALF), lambda j: (0, 0)),
        pl.BlockSpec((D_MODEL, D_MODEL), lambda j: (0, 0)),
        pl.BlockSpec((1, D_MODEL), lambda j: (0, 0)),
    ]
    if alias:
        in_specs.append(pl.BlockSpec(memory_space=pl.ANY))
    return pl.pallas_call(
        _tc_body,
        grid=grid,
        in_specs=in_specs,
        out_specs=pl.BlockSpec((1, b, D_MODEL), lambda j: (base + j, 0, 0)),
        out_shape=jax.ShapeDtypeStruct((l, b, D_MODEL), jnp.float32),
        input_output_aliases={6: 0} if alias else {},
    )


def kernel(card_ids, card_stats, emb_table, W_stat, b_stat, W_comb, b_comb):
    b, l = card_ids.shape
    n_rows = b * l
    ns = N_SPLIT
    rows_q = n_rows // ns

    flat_ids = jnp.transpose(card_ids, (1, 0)).reshape(n_rows)
    flat_ids = flat_ids.astype(jnp.int32) * 2
    table2 = jnp.pad(emb_table, ((0, 0), (0, D_MODEL - D_HALF)))
    table2 = table2.reshape(2 * table2.shape[0], D_HALF)

    gather = _make_sc_gather(rows_q, 640)
    id_emb_q = [
        gather(lax.slice(flat_ids, (q * rows_q,), ((q + 1) * rows_q,)), table2)
        for q in range(ns)
    ]

    st_j = jnp.transpose(card_stats, (1, 2, 0))
    bst = b_stat.reshape(1, D_HALF)
    bc = b_comb.reshape(1, D_MODEL)
    out = _make_tc_dense(b, l, 0, ns, False)(
        id_emb_q[0], st_j, W_stat, bst, W_comb, bc)
    for q in range(1, ns):
        out = _make_tc_dense(b, l, q, ns, True)(
            id_emb_q[q], st_j, W_stat, bst, W_comb, bc, out)
    return jnp.transpose(out, (1, 0, 2))
